# baseline (speedup 1.0000x reference)
import functools

import jax
import jax.numpy as jnp
from jax import lax
from jax.experimental import pallas as pl
from jax.experimental.pallas import tpu as pltpu
from jax.experimental.pallas import tpu_sc as plsc

_D = 128

_mesh = plsc.VectorSubcoreMesh(core_axis_name="c", subcore_axis_name="s")


@functools.partial(
    pl.kernel,
    mesh=_mesh,
    out_type=jax.ShapeDtypeStruct((1, _D), jnp.float32),
    scratch_types=[
        pltpu.VMEM((1,), jnp.int32),
        pltpu.VMEM((1, _D), jnp.float32),
        pltpu.SemaphoreType.DMA,
    ],
)
def _gather_row(x_hbm, emb_hbm, out_hbm, idx_v, row_v, sem):
    wid = lax.axis_index("s") * 2 + lax.axis_index("c")

    @pl.when(wid == 0)
    def _():
        pltpu.sync_copy(x_hbm, idx_v)
        pltpu.async_copy(emb_hbm.at[idx_v], row_v, sem).wait()
        pltpu.sync_copy(row_v, out_hbm)


def kernel(x, emb):
    return _gather_row(x.astype(jnp.int32), emb).reshape(64, 2)


# speedup vs baseline: 1.0876x; 1.0876x over previous
import functools

import jax
import jax.numpy as jnp
from jax import lax
from jax.experimental import pallas as pl
from jax.experimental.pallas import tpu as pltpu
from jax.experimental.pallas import tpu_sc as plsc

_D = 128

_mesh = plsc.VectorSubcoreMesh(
    core_axis_name="c", subcore_axis_name="s", num_cores=1, num_subcores=1
)


@functools.partial(
    pl.kernel,
    mesh=_mesh,
    out_type=jax.ShapeDtypeStruct((1, _D), jnp.float32),
    scratch_types=[
        pltpu.VMEM((1,), jnp.int32),
        pltpu.VMEM((1, _D), jnp.float32),
        pltpu.SemaphoreType.DMA,
    ],
)
def _gather_row(x_hbm, emb_hbm, out_hbm, idx_v, row_v, sem):
    pltpu.sync_copy(x_hbm, idx_v)
    pltpu.async_copy(emb_hbm.at[idx_v], row_v, sem).wait()
    pltpu.sync_copy(row_v, out_hbm)


def kernel(x, emb):
    return _gather_row(x.astype(jnp.int32), emb).reshape(64, 2)


# speedup vs baseline: 1.1609x; 1.0674x over previous
import functools

import jax
import jax.numpy as jnp
from jax.experimental import pallas as pl
from jax.experimental.pallas import tpu as pltpu
from jax.experimental.pallas import tpu_sc as plsc

_D = 128

_mesh = plsc.ScalarSubcoreMesh(axis_name="c", num_cores=1)


@functools.partial(
    pl.kernel,
    mesh=_mesh,
    out_type=jax.ShapeDtypeStruct((1, _D), jnp.float32),
    scratch_types=[
        pltpu.SMEM((1,), jnp.int32),
    ],
)
def _gather_row(x_hbm, emb_hbm, out_hbm, idx_s):
    pltpu.sync_copy(x_hbm, idx_s)
    i = idx_s[0]
    pltpu.sync_copy(emb_hbm.at[pl.ds(i, 1)], out_hbm)


def kernel(x, emb):
    return _gather_row(x.astype(jnp.int32), emb).reshape(64, 2)
